# baseline (speedup 1.0000x reference)
import functools

import jax
import jax.numpy as jnp
from jax import lax
from jax.experimental import pallas as pl
from jax.experimental.pallas import tpu as pltpu
from jax.experimental.pallas import tpu_sc as plsc

_NC = 2
_NS = 16
_F32 = jnp.float32
_HIGH = lax.Precision.HIGHEST


def _dot(a, b):
    return jnp.dot(a, b, preferred_element_type=_F32, precision=_HIGH)


def _sc_degree(dst, ones_rows, zeros16):
    E = dst.shape[0]
    N = zeros16.shape[0] * _NS
    W = zeros16.shape[1]
    NW = _NC * _NS
    epw = E // NW
    C = 128
    nfull = epw // C
    tail = epw - nfull * C
    rpt = N // _NS
    mesh = plsc.VectorSubcoreMesh(core_axis_name="c", subcore_axis_name="s")

    @functools.partial(
        pl.kernel, mesh=mesh,
        out_type=jax.ShapeDtypeStruct((_NC, N, W), _F32),
        scratch_types=[
            pltpu.VMEM((C,), jnp.int32),
            pltpu.VMEM((tail,), jnp.int32),
            pltpu.VMEM((C, W), _F32),
            pltpu.VMEM_SHARED((N, W), _F32),
        ],
    )
    def k(dst_hbm, ones_hbm, zero_hbm, out_hbm, dstv, dstv2, onesv, acc):
        c = lax.axis_index("c")
        s = lax.axis_index("s")
        wid = c * _NS + s
        r0 = s * rpt
        pltpu.sync_copy(ones_hbm, onesv)
        pltpu.sync_copy(zero_hbm, acc.at[pl.ds(r0, rpt)])
        plsc.subcore_barrier()
        base = wid * epw

        @pl.loop(0, nfull)
        def _(i):
            pltpu.sync_copy(dst_hbm.at[pl.ds(base + i * C, C)], dstv)
            pltpu.sync_copy(onesv, acc.at[dstv], add=True)

        if tail:
            pltpu.sync_copy(dst_hbm.at[pl.ds(base + nfull * C, tail)], dstv2)
            pltpu.sync_copy(onesv.at[pl.ds(0, tail)], acc.at[dstv2], add=True)
        plsc.subcore_barrier()
        pltpu.sync_copy(acc.at[pl.ds(r0, rpt)], out_hbm.at[c, pl.ds(r0, rpt)])

    return k(dst, ones_rows, zeros16)


_C = 128
_B = 8


def _sc_propagate(u, idx3, zeros_h):
    H = u.shape[1]
    N = zeros_h.shape[0] * _NS
    NW, nch2, C = idx3.shape
    nch = nch2 // 2
    rpt = N // _NS
    B = _B
    assert nch % B == 0 and B % 2 == 0
    mesh = plsc.VectorSubcoreMesh(core_axis_name="c", subcore_axis_name="s")

    @functools.partial(
        pl.kernel, mesh=mesh,
        out_type=jax.ShapeDtypeStruct((_NC, N, H), _F32),
        scratch_types=[
            pltpu.VMEM((2 * B, C), jnp.int32),
            pltpu.VMEM((2, C, H), _F32),
            pltpu.VMEM_SHARED((N, H), _F32),
            pltpu.SemaphoreType.DMA,
            pltpu.SemaphoreType.DMA,
        ],
    )
    def k(u_hbm, idx_hbm, zero_hbm, out_hbm, idxv, rows, acc, gs0, gs1):
        cc = lax.axis_index("c")
        s = lax.axis_index("s")
        wid = cc * _NS + s
        r0 = s * rpt

        pltpu.sync_copy(zero_hbm, acc.at[pl.ds(r0, rpt)])
        plsc.subcore_barrier()

        @pl.loop(0, nch // 2)
        def _(i):
            c0 = 2 * i
            j = 2 * (c0 % B)

            @pl.when(c0 % B == 0)
            def _():
                pltpu.sync_copy(
                    idx_hbm.at[wid, pl.ds((c0 // B) * 2 * B, 2 * B)], idxv)

            d0 = pltpu.async_copy(u_hbm.at[idxv.at[j]], rows.at[0], gs0)
            d1 = pltpu.async_copy(u_hbm.at[idxv.at[j + 2]], rows.at[1], gs1)
            d0.wait()
            pltpu.sync_copy(rows.at[0], acc.at[idxv.at[j + 1]], add=True)
            d1.wait()
            pltpu.sync_copy(rows.at[1], acc.at[idxv.at[j + 3]], add=True)

        plsc.subcore_barrier()
        pltpu.sync_copy(acc.at[pl.ds(r0, rpt)], out_hbm.at[cc, pl.ds(r0, rpt)])

    return k(u, idx3, zeros_h)


_BLK = 400


def _tc_encoder(x, W, b):
    N, F = x.shape
    H = W.shape[1]

    def body(x_ref, w_ref, b_ref, o_ref):
        o_ref[...] = jnp.maximum(_dot(x_ref[...], w_ref[...]) + b_ref[...], 0.0)

    return pl.pallas_call(
        body,
        grid=(N // _BLK,),
        in_specs=[
            pl.BlockSpec((_BLK, F), lambda i: (i, 0)),
            pl.BlockSpec((F, H), lambda i: (0, 0)),
            pl.BlockSpec((1, H), lambda i: (0, 0)),
        ],
        out_specs=pl.BlockSpec((_BLK, H), lambda i: (i, 0)),
        out_shape=jax.ShapeDtypeStruct((N, H), _F32),
    )(x, W, b.reshape(1, H))


def _tc_prep(degp, h0):
    N, H = h0.shape

    def body(d_ref, h_ref, dinv_ref, u_ref):
        deg = d_ref[0, :, :1] + d_ref[1, :, :1]
        dinv = jnp.where(deg > 0.0, lax.rsqrt(deg), 0.0)
        dinv_b = jnp.broadcast_to(dinv, (_BLK, H))
        dinv_ref[...] = dinv_b
        u_ref[...] = dinv_b * h_ref[...]

    return pl.pallas_call(
        body,
        grid=(N // _BLK,),
        in_specs=[
            pl.BlockSpec((2, _BLK, H), lambda i: (0, i, 0)),
            pl.BlockSpec((_BLK, H), lambda i: (i, 0)),
        ],
        out_specs=[
            pl.BlockSpec((_BLK, H), lambda i: (i, 0)),
            pl.BlockSpec((_BLK, H), lambda i: (i, 0)),
        ],
        out_shape=[
            jax.ShapeDtypeStruct((N, H), _F32),
            jax.ShapeDtypeStruct((N, H), _F32),
        ],
    )(degp, h0)


def _tc_upass(parts, dinv):
    _, N, H = parts.shape

    def body(p_ref, d_ref, u_ref):
        d = d_ref[...]
        u_ref[...] = d * d * (p_ref[0] + p_ref[1])

    return pl.pallas_call(
        body,
        grid=(N // _BLK,),
        in_specs=[
            pl.BlockSpec((2, _BLK, H), lambda i: (0, i, 0)),
            pl.BlockSpec((_BLK, H), lambda i: (i, 0)),
        ],
        out_specs=pl.BlockSpec((_BLK, H), lambda i: (i, 0)),
        out_shape=jax.ShapeDtypeStruct((N, H), _F32),
    )(parts, dinv)


def _tc_conv(h, p1, p2, p3, dinv, Wstack, bias):
    N, H = h.shape
    ngrid = N // _BLK

    def body(h_ref, p1_ref, p2_ref, p3_ref, d_ref, w_ref, b_ref,
             raw_ref, sum_ref, sq_ref, acc_s, acc_q):
        i = pl.program_id(0)

        @pl.when(i == 0)
        def _():
            acc_s[...] = jnp.zeros_like(acc_s)
            acc_q[...] = jnp.zeros_like(acc_q)

        d = d_ref[...]
        h1 = d * (p1_ref[0] + p1_ref[1])
        h2 = d * (p2_ref[0] + p2_ref[1])
        h3 = d * (p3_ref[0] + p3_ref[1])
        out = (_dot(h_ref[...], w_ref[0]) + _dot(h1, w_ref[1])
               + _dot(h2, w_ref[2]) + _dot(h3, w_ref[3]) + b_ref[...])
        raw_ref[...] = out
        acc_s[...] += jnp.sum(out, axis=0, keepdims=True)
        acc_q[...] += jnp.sum(out * out, axis=0, keepdims=True)

        @pl.when(i == ngrid - 1)
        def _():
            sum_ref[...] = acc_s[...]
            sq_ref[...] = acc_q[...]

    return pl.pallas_call(
        body,
        grid=(ngrid,),
        in_specs=[
            pl.BlockSpec((_BLK, H), lambda i: (i, 0)),
            pl.BlockSpec((2, _BLK, H), lambda i: (0, i, 0)),
            pl.BlockSpec((2, _BLK, H), lambda i: (0, i, 0)),
            pl.BlockSpec((2, _BLK, H), lambda i: (0, i, 0)),
            pl.BlockSpec((_BLK, H), lambda i: (i, 0)),
            pl.BlockSpec((4, H, H), lambda i: (0, 0, 0)),
            pl.BlockSpec((1, H), lambda i: (0, 0)),
        ],
        out_specs=[
            pl.BlockSpec((_BLK, H), lambda i: (i, 0)),
            pl.BlockSpec((1, H), lambda i: (0, 0)),
            pl.BlockSpec((1, H), lambda i: (0, 0)),
        ],
        out_shape=[
            jax.ShapeDtypeStruct((N, H), _F32),
            jax.ShapeDtypeStruct((1, H), _F32),
            jax.ShapeDtypeStruct((1, H), _F32),
        ],
        scratch_shapes=[
            pltpu.VMEM((1, H), _F32),
            pltpu.VMEM((1, H), _F32),
        ],
    )(h, p1, p2, p3, dinv, Wstack, bias.reshape(1, H))


def _tc_bn(raw, ssum, ssq, gamma, beta, dinv, want_u):
    N, H = raw.shape
    inv_n = 1.0 / N

    def body(r_ref, s_ref, q_ref, g_ref, be_ref, d_ref, *outs):
        m = s_ref[...] * inv_n
        v = q_ref[...] * inv_n - m * m
        hn = (r_ref[...] - m) * lax.rsqrt(v + 1e-5) * g_ref[...] + be_ref[...]
        hn = jnp.maximum(hn, 0.0)
        outs[0][...] = hn
        if want_u:
            outs[1][...] = d_ref[...] * hn

    nout = 2 if want_u else 1
    return pl.pallas_call(
        body,
        grid=(N // _BLK,),
        in_specs=[
            pl.BlockSpec((_BLK, H), lambda i: (i, 0)),
            pl.BlockSpec((1, H), lambda i: (0, 0)),
            pl.BlockSpec((1, H), lambda i: (0, 0)),
            pl.BlockSpec((1, H), lambda i: (0, 0)),
            pl.BlockSpec((1, H), lambda i: (0, 0)),
            pl.BlockSpec((_BLK, H), lambda i: (i, 0)),
        ],
        out_specs=[pl.BlockSpec((_BLK, H), lambda i: (i, 0))] * nout,
        out_shape=[jax.ShapeDtypeStruct((N, H), _F32)] * nout,
    )(raw, ssum, ssq, gamma.reshape(1, H), beta.reshape(1, H), dinv)


def _tc_pool(h, bidx3, G):
    N, H = h.shape
    ngrid = N // _BLK

    def body(h_ref, b_ref, xs_ref, cn_ref, acc_x, acc_c):
        i = pl.program_id(0)

        @pl.when(i == 0)
        def _():
            acc_x[...] = jnp.zeros_like(acc_x)
            acc_c[...] = jnp.zeros_like(acc_c)

        b = b_ref[0, 0, :]
        gids = lax.broadcasted_iota(jnp.int32, (_BLK, G), 1)
        mask = (b[:, None] == gids).astype(_F32)
        cn = lax.dot_general(mask, jnp.ones((_BLK, H), _F32),
                             (((0,), (0,)), ((), ())),
                             preferred_element_type=_F32, precision=_HIGH)
        xs = lax.dot_general(mask, h_ref[...], (((0,), (0,)), ((), ())),
                             preferred_element_type=_F32, precision=_HIGH)
        acc_x[...] += xs
        acc_c[...] += cn

        @pl.when(i == ngrid - 1)
        def _():
            xs_ref[...] = acc_x[...]
            cn_ref[...] = acc_c[...]

    return pl.pallas_call(
        body,
        grid=(ngrid,),
        in_specs=[
            pl.BlockSpec((_BLK, H), lambda i: (i, 0)),
            pl.BlockSpec((1, 1, _BLK), lambda i: (i, 0, 0)),
        ],
        out_specs=[
            pl.BlockSpec((G, H), lambda i: (0, 0)),
            pl.BlockSpec((G, H), lambda i: (0, 0)),
        ],
        out_shape=[
            jax.ShapeDtypeStruct((G, H), _F32),
            jax.ShapeDtypeStruct((G, H), _F32),
        ],
        scratch_shapes=[
            pltpu.VMEM((G, H), _F32),
            pltpu.VMEM((G, H), _F32),
        ],
    )(h, bidx3)


def _tc_head(xsum, counts, latf_pad, Wlat_pad, b_lat,
             W1m, W1s, W1l, b1, W2, b2, W3_pad):
    G, H = xsum.shape
    H2 = W2.shape[1]

    def body(xs_ref, cn_ref, lf_ref, wl_ref, bl_ref, w1m_ref, w1s_ref,
             w1l_ref, b1_ref, w2_ref, b2_ref, w3_ref, o_ref):
        xs = xs_ref[...]
        cm = jnp.maximum(cn_ref[...], 1.0)
        xmean = xs / cm
        lat = jnp.maximum(_dot(lf_ref[...], wl_ref[...]) + bl_ref[...], 0.0)
        z = (_dot(xmean, w1m_ref[...]) + _dot(xs, w1s_ref[...])
             + _dot(lat, w1l_ref[...]) + b1_ref[...])
        z = jnp.maximum(z, 0.0)
        z = jnp.maximum(_dot(z, w2_ref[...]) + b2_ref[...], 0.0)
        o_ref[...] = _dot(z, w3_ref[...])

    return pl.pallas_call(
        body,
        out_shape=jax.ShapeDtypeStruct((G, H), _F32),
    )(xsum, counts, latf_pad, Wlat_pad, b_lat.reshape(1, H),
      W1m, W1s, W1l, b1.reshape(1, H), W2, b2.reshape(1, H2), W3_pad)


def kernel(x, edge_index, batch_idx, lattice_features, W_enc, b_enc, W_lat,
           b_lat, Wc1, bc1, g1, be1, Wc2, bc2, g2, be2, Wc3, bc3, g3, be3,
           W1, b1, W2, b2, W3, b3):
    N, F = x.shape
    H = W_enc.shape[1]
    G = lattice_features.shape[0]
    src = edge_index[0]
    dst = edge_index[1]
    npad = 8 * _NS
    Np = ((N + npad - 1) // npad) * npad
    rpt = Np // _NS

    ones_rows = jnp.ones((128, H), _F32)
    zeros_h = jnp.zeros((rpt, H), _F32)
    bidx3 = batch_idx.reshape(N // _BLK, 1, _BLK)

    E = src.shape[0]
    C = _C
    NW = _NC * _NS
    nch = -(-E // (NW * C))
    nch = max(((nch + _B - 1) // _B) * _B, _B)
    E_pad = NW * nch * C
    src_p = jnp.concatenate([src, jnp.zeros((E_pad - E,), jnp.int32)])
    dst_p = jnp.concatenate([dst, jnp.full((E_pad - E,), Np - 1, jnp.int32)])
    idx3 = (jnp.stack([src_p, dst_p], 0).reshape(2, NW, nch, C)
            .transpose(1, 2, 0, 3).reshape(NW, 2 * nch, C))

    degp = _sc_degree(dst, ones_rows, zeros_h)
    h = _tc_encoder(x, W_enc, b_enc)
    dinv, u = _tc_prep(degp, h)

    for Wc, bc, g, be, last in ((Wc1, bc1, g1, be1, False),
                                (Wc2, bc2, g2, be2, False),
                                (Wc3, bc3, g3, be3, True)):
        parts = []
        for k in range(3):
            p = _sc_propagate(u, idx3, zeros_h)
            parts.append(p)
            if k < 2:
                u = _tc_upass(p, dinv)
        raw, ssum, ssq = _tc_conv(h, parts[0], parts[1], parts[2], dinv, Wc, bc)
        if last:
            (h,) = _tc_bn(raw, ssum, ssq, g, be, dinv, want_u=False)
        else:
            h, u = _tc_bn(raw, ssum, ssq, g, be, dinv, want_u=True)

    xsum, counts = _tc_pool(h, bidx3, G)

    latf_pad = jnp.pad(lattice_features, ((0, 0), (0, 16 - lattice_features.shape[1])))
    Wlat_pad = jnp.pad(W_lat, ((0, 16 - W_lat.shape[0]), (0, 0)))
    W3_pad = jnp.pad(W3, ((0, 0), (0, H - W3.shape[1])))
    W1m, W1s, W1l = W1[:H], W1[H:2 * H], W1[2 * H:]

    head = _tc_head(xsum, counts, latf_pad, Wlat_pad, b_lat,
                    W1m, W1s, W1l, b1, W2, b2, W3_pad)
    return head[:, 0] + b3[0]


# speedup vs baseline: 2.1510x; 2.1510x over previous
import functools

import jax
import jax.numpy as jnp
from jax import lax
from jax.experimental import pallas as pl
from jax.experimental.pallas import tpu as pltpu
from jax.experimental.pallas import tpu_sc as plsc

_NC = 2
_NS = 16
_F32 = jnp.float32
_HIGH = lax.Precision.HIGHEST


def _dot(a, b):
    return jnp.dot(a, b, preferred_element_type=_F32, precision=_HIGH)


def _sc_degree(dst, ones_rows, zeros16):
    E = dst.shape[0]
    N = zeros16.shape[0] * _NS
    W = zeros16.shape[1]
    NW = _NC * _NS
    epw = E // NW
    C = 128
    nfull = epw // C
    tail = epw - nfull * C
    rpt = N // _NS
    mesh = plsc.VectorSubcoreMesh(core_axis_name="c", subcore_axis_name="s")

    @functools.partial(
        pl.kernel, mesh=mesh,
        out_type=jax.ShapeDtypeStruct((_NC, N, W), _F32),
        scratch_types=[
            pltpu.VMEM((C,), jnp.int32),
            pltpu.VMEM((tail,), jnp.int32),
            pltpu.VMEM((C, W), _F32),
            pltpu.VMEM_SHARED((N, W), _F32),
        ],
    )
    def k(dst_hbm, ones_hbm, zero_hbm, out_hbm, dstv, dstv2, onesv, acc):
        c = lax.axis_index("c")
        s = lax.axis_index("s")
        wid = c * _NS + s
        r0 = s * rpt
        pltpu.sync_copy(ones_hbm, onesv)
        pltpu.sync_copy(zero_hbm, acc.at[pl.ds(r0, rpt)])
        plsc.subcore_barrier()
        base = wid * epw

        @pl.loop(0, nfull)
        def _(i):
            pltpu.sync_copy(dst_hbm.at[pl.ds(base + i * C, C)], dstv)
            pltpu.sync_copy(onesv, acc.at[dstv], add=True)

        if tail:
            pltpu.sync_copy(dst_hbm.at[pl.ds(base + nfull * C, tail)], dstv2)
            pltpu.sync_copy(onesv.at[pl.ds(0, tail)], acc.at[dstv2], add=True)
        plsc.subcore_barrier()
        pltpu.sync_copy(acc.at[pl.ds(r0, rpt)], out_hbm.at[c, pl.ds(r0, rpt)])

    return k(dst, ones_rows, zeros16)


def _sc_propagate(u, src, dst, zeros_h):
    H = u.shape[1]
    N = zeros_h.shape[0] * _NS
    E = src.shape[0]
    NW = _NC * _NS
    epw = E // NW
    C = 128
    npair = epw // (2 * C)
    tail = epw - npair * 2 * C
    rpt = N // _NS
    mesh = plsc.VectorSubcoreMesh(core_axis_name="c", subcore_axis_name="s")

    @functools.partial(
        pl.kernel, mesh=mesh,
        out_type=jax.ShapeDtypeStruct((_NC, N, H), _F32),
        scratch_types=[
            pltpu.VMEM((C,), jnp.int32),
            pltpu.VMEM((C,), jnp.int32),
            pltpu.VMEM((C,), jnp.int32),
            pltpu.VMEM((C,), jnp.int32),
            pltpu.VMEM((C, H), _F32),
            pltpu.VMEM((C, H), _F32),
            pltpu.VMEM((tail,), jnp.int32),
            pltpu.VMEM((tail,), jnp.int32),
            pltpu.VMEM((tail, H), _F32),
            pltpu.VMEM_SHARED((N, H), _F32),
            pltpu.SemaphoreType.DMA,
            pltpu.SemaphoreType.DMA,
        ],
    )
    def k(u_hbm, src_hbm, dst_hbm, zero_hbm, out_hbm,
          srcv0, dstv0, srcv1, dstv1, rows0, rows1,
          srcv2, dstv2, rowsv2, acc, sem0, sem1):
        c = lax.axis_index("c")
        s = lax.axis_index("s")
        wid = c * _NS + s
        r0 = s * rpt
        pltpu.sync_copy(zero_hbm, acc.at[pl.ds(r0, rpt)])
        plsc.subcore_barrier()
        base = wid * epw

        @pl.loop(0, npair)
        def _(i):
            off0 = base + (2 * i) * C
            off1 = off0 + C
            pltpu.sync_copy(src_hbm.at[pl.ds(off0, C)], srcv0)
            pltpu.sync_copy(dst_hbm.at[pl.ds(off0, C)], dstv0)
            pltpu.sync_copy(src_hbm.at[pl.ds(off1, C)], srcv1)
            pltpu.sync_copy(dst_hbm.at[pl.ds(off1, C)], dstv1)
            d0 = pltpu.async_copy(u_hbm.at[srcv0], rows0, sem0)
            d1 = pltpu.async_copy(u_hbm.at[srcv1], rows1, sem1)
            d0.wait()
            pltpu.sync_copy(rows0, acc.at[dstv0], add=True)
            d1.wait()
            pltpu.sync_copy(rows1, acc.at[dstv1], add=True)

        if tail:
            off = base + npair * 2 * C
            pltpu.sync_copy(src_hbm.at[pl.ds(off, tail)], srcv2)
            pltpu.sync_copy(dst_hbm.at[pl.ds(off, tail)], dstv2)
            pltpu.async_copy(u_hbm.at[srcv2], rowsv2, sem0).wait()
            pltpu.sync_copy(rowsv2, acc.at[dstv2], add=True)
        plsc.subcore_barrier()
        pltpu.sync_copy(acc.at[pl.ds(r0, rpt)], out_hbm.at[c, pl.ds(r0, rpt)])

    return k(u, src, dst, zeros_h)


_BLK = 400


def _tc_encoder(x, W, b):
    N, F = x.shape
    H = W.shape[1]

    def body(x_ref, w_ref, b_ref, o_ref):
        o_ref[...] = jnp.maximum(_dot(x_ref[...], w_ref[...]) + b_ref[...], 0.0)

    return pl.pallas_call(
        body,
        grid=(N // _BLK,),
        in_specs=[
            pl.BlockSpec((_BLK, F), lambda i: (i, 0)),
            pl.BlockSpec((F, H), lambda i: (0, 0)),
            pl.BlockSpec((1, H), lambda i: (0, 0)),
        ],
        out_specs=pl.BlockSpec((_BLK, H), lambda i: (i, 0)),
        out_shape=jax.ShapeDtypeStruct((N, H), _F32),
    )(x, W, b.reshape(1, H))


def _tc_prep(degp, h0):
    N, H = h0.shape

    def body(d_ref, h_ref, dinv_ref, u_ref):
        deg = d_ref[0, :, :1] + d_ref[1, :, :1]
        dinv = jnp.where(deg > 0.0, lax.rsqrt(deg), 0.0)
        dinv_b = jnp.broadcast_to(dinv, (_BLK, H))
        dinv_ref[...] = dinv_b
        u_ref[...] = dinv_b * h_ref[...]

    return pl.pallas_call(
        body,
        grid=(N // _BLK,),
        in_specs=[
            pl.BlockSpec((2, _BLK, H), lambda i: (0, i, 0)),
            pl.BlockSpec((_BLK, H), lambda i: (i, 0)),
        ],
        out_specs=[
            pl.BlockSpec((_BLK, H), lambda i: (i, 0)),
            pl.BlockSpec((_BLK, H), lambda i: (i, 0)),
        ],
        out_shape=[
            jax.ShapeDtypeStruct((N, H), _F32),
            jax.ShapeDtypeStruct((N, H), _F32),
        ],
    )(degp, h0)


def _tc_upass(parts, dinv):
    _, N, H = parts.shape

    def body(p_ref, d_ref, u_ref):
        d = d_ref[...]
        u_ref[...] = d * d * (p_ref[0] + p_ref[1])

    return pl.pallas_call(
        body,
        grid=(N // _BLK,),
        in_specs=[
            pl.BlockSpec((2, _BLK, H), lambda i: (0, i, 0)),
            pl.BlockSpec((_BLK, H), lambda i: (i, 0)),
        ],
        out_specs=pl.BlockSpec((_BLK, H), lambda i: (i, 0)),
        out_shape=jax.ShapeDtypeStruct((N, H), _F32),
    )(parts, dinv)


def _tc_conv(h, p1, p2, p3, dinv, Wstack, bias):
    N, H = h.shape
    ngrid = N // _BLK

    def body(h_ref, p1_ref, p2_ref, p3_ref, d_ref, w_ref, b_ref,
             raw_ref, sum_ref, sq_ref, acc_s, acc_q):
        i = pl.program_id(0)

        @pl.when(i == 0)
        def _():
            acc_s[...] = jnp.zeros_like(acc_s)
            acc_q[...] = jnp.zeros_like(acc_q)

        d = d_ref[...]
        h1 = d * (p1_ref[0] + p1_ref[1])
        h2 = d * (p2_ref[0] + p2_ref[1])
        h3 = d * (p3_ref[0] + p3_ref[1])
        out = (_dot(h_ref[...], w_ref[0]) + _dot(h1, w_ref[1])
               + _dot(h2, w_ref[2]) + _dot(h3, w_ref[3]) + b_ref[...])
        raw_ref[...] = out
        acc_s[...] += jnp.sum(out, axis=0, keepdims=True)
        acc_q[...] += jnp.sum(out * out, axis=0, keepdims=True)

        @pl.when(i == ngrid - 1)
        def _():
            sum_ref[...] = acc_s[...]
            sq_ref[...] = acc_q[...]

    return pl.pallas_call(
        body,
        grid=(ngrid,),
        in_specs=[
            pl.BlockSpec((_BLK, H), lambda i: (i, 0)),
            pl.BlockSpec((2, _BLK, H), lambda i: (0, i, 0)),
            pl.BlockSpec((2, _BLK, H), lambda i: (0, i, 0)),
            pl.BlockSpec((2, _BLK, H), lambda i: (0, i, 0)),
            pl.BlockSpec((_BLK, H), lambda i: (i, 0)),
            pl.BlockSpec((4, H, H), lambda i: (0, 0, 0)),
            pl.BlockSpec((1, H), lambda i: (0, 0)),
        ],
        out_specs=[
            pl.BlockSpec((_BLK, H), lambda i: (i, 0)),
            pl.BlockSpec((1, H), lambda i: (0, 0)),
            pl.BlockSpec((1, H), lambda i: (0, 0)),
        ],
        out_shape=[
            jax.ShapeDtypeStruct((N, H), _F32),
            jax.ShapeDtypeStruct((1, H), _F32),
            jax.ShapeDtypeStruct((1, H), _F32),
        ],
        scratch_shapes=[
            pltpu.VMEM((1, H), _F32),
            pltpu.VMEM((1, H), _F32),
        ],
    )(h, p1, p2, p3, dinv, Wstack, bias.reshape(1, H))


def _tc_bn(raw, ssum, ssq, gamma, beta, dinv, want_u):
    N, H = raw.shape
    inv_n = 1.0 / N

    def body(r_ref, s_ref, q_ref, g_ref, be_ref, d_ref, *outs):
        m = s_ref[...] * inv_n
        v = q_ref[...] * inv_n - m * m
        hn = (r_ref[...] - m) * lax.rsqrt(v + 1e-5) * g_ref[...] + be_ref[...]
        hn = jnp.maximum(hn, 0.0)
        outs[0][...] = hn
        if want_u:
            outs[1][...] = d_ref[...] * hn

    nout = 2 if want_u else 1
    return pl.pallas_call(
        body,
        grid=(N // _BLK,),
        in_specs=[
            pl.BlockSpec((_BLK, H), lambda i: (i, 0)),
            pl.BlockSpec((1, H), lambda i: (0, 0)),
            pl.BlockSpec((1, H), lambda i: (0, 0)),
            pl.BlockSpec((1, H), lambda i: (0, 0)),
            pl.BlockSpec((1, H), lambda i: (0, 0)),
            pl.BlockSpec((_BLK, H), lambda i: (i, 0)),
        ],
        out_specs=[pl.BlockSpec((_BLK, H), lambda i: (i, 0))] * nout,
        out_shape=[jax.ShapeDtypeStruct((N, H), _F32)] * nout,
    )(raw, ssum, ssq, gamma.reshape(1, H), beta.reshape(1, H), dinv)


def _tc_pool(h, bidx3, G):
    N, H = h.shape
    ngrid = N // _BLK

    def body(h_ref, b_ref, xs_ref, cn_ref, acc_x, acc_c):
        i = pl.program_id(0)

        @pl.when(i == 0)
        def _():
            acc_x[...] = jnp.zeros_like(acc_x)
            acc_c[...] = jnp.zeros_like(acc_c)

        b = b_ref[0, 0, :]
        gids = lax.broadcasted_iota(jnp.int32, (_BLK, G), 1)
        mask = (b[:, None] == gids).astype(_F32)
        cn = lax.dot_general(mask, jnp.ones((_BLK, H), _F32),
                             (((0,), (0,)), ((), ())),
                             preferred_element_type=_F32, precision=_HIGH)
        xs = lax.dot_general(mask, h_ref[...], (((0,), (0,)), ((), ())),
                             preferred_element_type=_F32, precision=_HIGH)
        acc_x[...] += xs
        acc_c[...] += cn

        @pl.when(i == ngrid - 1)
        def _():
            xs_ref[...] = acc_x[...]
            cn_ref[...] = acc_c[...]

    return pl.pallas_call(
        body,
        grid=(ngrid,),
        in_specs=[
            pl.BlockSpec((_BLK, H), lambda i: (i, 0)),
            pl.BlockSpec((1, 1, _BLK), lambda i: (i, 0, 0)),
        ],
        out_specs=[
            pl.BlockSpec((G, H), lambda i: (0, 0)),
            pl.BlockSpec((G, H), lambda i: (0, 0)),
        ],
        out_shape=[
            jax.ShapeDtypeStruct((G, H), _F32),
            jax.ShapeDtypeStruct((G, H), _F32),
        ],
        scratch_shapes=[
            pltpu.VMEM((G, H), _F32),
            pltpu.VMEM((G, H), _F32),
        ],
    )(h, bidx3)


def _tc_head(xsum, counts, latf_pad, Wlat_pad, b_lat,
             W1m, W1s, W1l, b1, W2, b2, W3_pad):
    G, H = xsum.shape
    H2 = W2.shape[1]

    def body(xs_ref, cn_ref, lf_ref, wl_ref, bl_ref, w1m_ref, w1s_ref,
             w1l_ref, b1_ref, w2_ref, b2_ref, w3_ref, o_ref):
        xs = xs_ref[...]
        cm = jnp.maximum(cn_ref[...], 1.0)
        xmean = xs / cm
        lat = jnp.maximum(_dot(lf_ref[...], wl_ref[...]) + bl_ref[...], 0.0)
        z = (_dot(xmean, w1m_ref[...]) + _dot(xs, w1s_ref[...])
             + _dot(lat, w1l_ref[...]) + b1_ref[...])
        z = jnp.maximum(z, 0.0)
        z = jnp.maximum(_dot(z, w2_ref[...]) + b2_ref[...], 0.0)
        o_ref[...] = _dot(z, w3_ref[...])

    return pl.pallas_call(
        body,
        out_shape=jax.ShapeDtypeStruct((G, H), _F32),
    )(xsum, counts, latf_pad, Wlat_pad, b_lat.reshape(1, H),
      W1m, W1s, W1l, b1.reshape(1, H), W2, b2.reshape(1, H2), W3_pad)


def kernel(x, edge_index, batch_idx, lattice_features, W_enc, b_enc, W_lat,
           b_lat, Wc1, bc1, g1, be1, Wc2, bc2, g2, be2, Wc3, bc3, g3, be3,
           W1, b1, W2, b2, W3, b3):
    N, F = x.shape
    H = W_enc.shape[1]
    G = lattice_features.shape[0]
    src = edge_index[0]
    dst = edge_index[1]
    npad = 8 * _NS
    Np = ((N + npad - 1) // npad) * npad
    rpt = Np // _NS

    ones_rows = jnp.ones((128, H), _F32)
    zeros_h = jnp.zeros((rpt, H), _F32)
    bidx3 = batch_idx.reshape(N // _BLK, 1, _BLK)

    degp = _sc_degree(dst, ones_rows, zeros_h)
    h = _tc_encoder(x, W_enc, b_enc)
    dinv, u = _tc_prep(degp, h)

    for Wc, bc, g, be, last in ((Wc1, bc1, g1, be1, False),
                                (Wc2, bc2, g2, be2, False),
                                (Wc3, bc3, g3, be3, True)):
        parts = []
        for k in range(3):
            p = _sc_propagate(u, src, dst, zeros_h)
            parts.append(p)
            if k < 2:
                u = _tc_upass(p, dinv)
        raw, ssum, ssq = _tc_conv(h, parts[0], parts[1], parts[2], dinv, Wc, bc)
        if last:
            (h,) = _tc_bn(raw, ssum, ssq, g, be, dinv, want_u=False)
        else:
            h, u = _tc_bn(raw, ssum, ssq, g, be, dinv, want_u=True)

    xsum, counts = _tc_pool(h, bidx3, G)

    latf_pad = jnp.pad(lattice_features, ((0, 0), (0, 16 - lattice_features.shape[1])))
    Wlat_pad = jnp.pad(W_lat, ((0, 16 - W_lat.shape[0]), (0, 0)))
    W3_pad = jnp.pad(W3, ((0, 0), (0, H - W3.shape[1])))
    W1m, W1s, W1l = W1[:H], W1[H:2 * H], W1[2 * H:]

    head = _tc_head(xsum, counts, latf_pad, Wlat_pad, b_lat,
                    W1m, W1s, W1l, b1, W2, b2, W3_pad)
    return head[:, 0] + b3[0]


# speedup vs baseline: 2.7265x; 1.2675x over previous
import functools

import jax
import jax.numpy as jnp
from jax import lax
from jax.experimental import pallas as pl
from jax.experimental.pallas import tpu as pltpu
from jax.experimental.pallas import tpu_sc as plsc

_NC = 2
_NS = 16
_F32 = jnp.float32
_HIGH = lax.Precision.HIGHEST


def _dot(a, b):
    return jnp.dot(a, b, preferred_element_type=_F32, precision=_HIGH)


def _sc_degree(dst, ones_rows, zeros16):
    E = dst.shape[0]
    N = zeros16.shape[0] * _NS
    W = zeros16.shape[1]
    NW = _NC * _NS
    epw = E // NW
    C = 128
    nfull = epw // C
    tail = epw - nfull * C
    rpt = N // _NS
    mesh = plsc.VectorSubcoreMesh(core_axis_name="c", subcore_axis_name="s")

    @functools.partial(
        pl.kernel, mesh=mesh,
        out_type=jax.ShapeDtypeStruct((_NC, N, W), _F32),
        scratch_types=[
            pltpu.VMEM((C,), jnp.int32),
            pltpu.VMEM((tail,), jnp.int32),
            pltpu.VMEM((C, W), _F32),
            pltpu.VMEM_SHARED((N, W), _F32),
        ],
    )
    def k(dst_hbm, ones_hbm, zero_hbm, out_hbm, dstv, dstv2, onesv, acc):
        c = lax.axis_index("c")
        s = lax.axis_index("s")
        wid = c * _NS + s
        r0 = s * rpt
        pltpu.sync_copy(ones_hbm, onesv)
        pltpu.sync_copy(zero_hbm, acc.at[pl.ds(r0, rpt)])
        plsc.subcore_barrier()
        base = wid * epw

        @pl.loop(0, nfull)
        def _(i):
            pltpu.sync_copy(dst_hbm.at[pl.ds(base + i * C, C)], dstv)
            pltpu.sync_copy(onesv, acc.at[dstv], add=True)

        if tail:
            pltpu.sync_copy(dst_hbm.at[pl.ds(base + nfull * C, tail)], dstv2)
            pltpu.sync_copy(onesv.at[pl.ds(0, tail)], acc.at[dstv2], add=True)
        plsc.subcore_barrier()
        pltpu.sync_copy(acc.at[pl.ds(r0, rpt)], out_hbm.at[c, pl.ds(r0, rpt)])

    return k(dst, ones_rows, zeros16)


def _sc_propagate(u, src, dst, zeros_h):
    H = u.shape[1]
    N = zeros_h.shape[0] * _NS
    E = src.shape[0]
    NW = _NC * _NS
    epw = E // NW
    C = 128
    npair = epw // (2 * C)
    tail = epw - npair * 2 * C
    rpt = N // _NS
    mesh = plsc.VectorSubcoreMesh(core_axis_name="c", subcore_axis_name="s")

    @functools.partial(
        pl.kernel, mesh=mesh,
        out_type=jax.ShapeDtypeStruct((_NC, N, H), _F32),
        scratch_types=[
            pltpu.VMEM((C,), jnp.int32),
            pltpu.VMEM((C,), jnp.int32),
            pltpu.VMEM((C,), jnp.int32),
            pltpu.VMEM((C,), jnp.int32),
            pltpu.VMEM((C, H), _F32),
            pltpu.VMEM((C, H), _F32),
            pltpu.VMEM((tail,), jnp.int32),
            pltpu.VMEM((tail,), jnp.int32),
            pltpu.VMEM((tail, H), _F32),
            pltpu.VMEM_SHARED((N, H), _F32),
            pltpu.SemaphoreType.DMA,
            pltpu.SemaphoreType.DMA,
            pltpu.SemaphoreType.DMA,
            pltpu.SemaphoreType.DMA,
            pltpu.SemaphoreType.DMA,
            pltpu.SemaphoreType.DMA,
        ],
    )
    def k(u_hbm, src_hbm, dst_hbm, zero_hbm, out_hbm,
          srcv0, dstv0, srcv1, dstv1, rows0, rows1,
          srcv2, dstv2, rowsv2, acc, sem0, sem1, is0, is1, is2, is3):
        c = lax.axis_index("c")
        s = lax.axis_index("s")
        wid = c * _NS + s
        r0 = s * rpt
        pltpu.sync_copy(zero_hbm, acc.at[pl.ds(r0, rpt)])
        plsc.subcore_barrier()
        base = wid * epw

        @pl.loop(0, npair)
        def _(i):
            off0 = base + (2 * i) * C
            off1 = off0 + C
            i0 = pltpu.async_copy(src_hbm.at[pl.ds(off0, C)], srcv0, is0)
            i1 = pltpu.async_copy(dst_hbm.at[pl.ds(off0, C)], dstv0, is1)
            i2 = pltpu.async_copy(src_hbm.at[pl.ds(off1, C)], srcv1, is2)
            i3 = pltpu.async_copy(dst_hbm.at[pl.ds(off1, C)], dstv1, is3)
            i0.wait()
            d0 = pltpu.async_copy(u_hbm.at[srcv0], rows0, sem0)
            i2.wait()
            d1 = pltpu.async_copy(u_hbm.at[srcv1], rows1, sem1)
            d0.wait()
            i1.wait()
            pltpu.sync_copy(rows0, acc.at[dstv0], add=True)
            d1.wait()
            i3.wait()
            pltpu.sync_copy(rows1, acc.at[dstv1], add=True)

        if tail:
            off = base + npair * 2 * C
            pltpu.sync_copy(src_hbm.at[pl.ds(off, tail)], srcv2)
            pltpu.sync_copy(dst_hbm.at[pl.ds(off, tail)], dstv2)
            pltpu.async_copy(u_hbm.at[srcv2], rowsv2, sem0).wait()
            pltpu.sync_copy(rowsv2, acc.at[dstv2], add=True)
        plsc.subcore_barrier()
        pltpu.sync_copy(acc.at[pl.ds(r0, rpt)], out_hbm.at[c, pl.ds(r0, rpt)])

    return k(u, src, dst, zeros_h)


_BLK = 400


def _tc_encoder(x, W, b):
    N, F = x.shape
    H = W.shape[1]

    def body(x_ref, w_ref, b_ref, o_ref):
        o_ref[...] = jnp.maximum(_dot(x_ref[...], w_ref[...]) + b_ref[...], 0.0)

    return pl.pallas_call(
        body,
        grid=(N // _BLK,),
        in_specs=[
            pl.BlockSpec((_BLK, F), lambda i: (i, 0)),
            pl.BlockSpec((F, H), lambda i: (0, 0)),
            pl.BlockSpec((1, H), lambda i: (0, 0)),
        ],
        out_specs=pl.BlockSpec((_BLK, H), lambda i: (i, 0)),
        out_shape=jax.ShapeDtypeStruct((N, H), _F32),
    )(x, W, b.reshape(1, H))


def _tc_prep(degp, h0):
    N, H = h0.shape

    def body(d_ref, h_ref, dinv_ref, u_ref):
        deg = d_ref[0, :, :1] + d_ref[1, :, :1]
        dinv = jnp.where(deg > 0.0, lax.rsqrt(deg), 0.0)
        dinv_b = jnp.broadcast_to(dinv, (_BLK, H))
        dinv_ref[...] = dinv_b
        u_ref[...] = dinv_b * h_ref[...]

    return pl.pallas_call(
        body,
        grid=(N // _BLK,),
        in_specs=[
            pl.BlockSpec((2, _BLK, H), lambda i: (0, i, 0)),
            pl.BlockSpec((_BLK, H), lambda i: (i, 0)),
        ],
        out_specs=[
            pl.BlockSpec((_BLK, H), lambda i: (i, 0)),
            pl.BlockSpec((_BLK, H), lambda i: (i, 0)),
        ],
        out_shape=[
            jax.ShapeDtypeStruct((N, H), _F32),
            jax.ShapeDtypeStruct((N, H), _F32),
        ],
    )(degp, h0)


def _tc_upass(parts, dinv):
    _, N, H = parts.shape

    def body(p_ref, d_ref, u_ref):
        d = d_ref[...]
        u_ref[...] = d * d * (p_ref[0] + p_ref[1])

    return pl.pallas_call(
        body,
        grid=(N // _BLK,),
        in_specs=[
            pl.BlockSpec((2, _BLK, H), lambda i: (0, i, 0)),
            pl.BlockSpec((_BLK, H), lambda i: (i, 0)),
        ],
        out_specs=pl.BlockSpec((_BLK, H), lambda i: (i, 0)),
        out_shape=jax.ShapeDtypeStruct((N, H), _F32),
    )(parts, dinv)


def _tc_conv(h, p1, p2, p3, dinv, Wstack, bias):
    N, H = h.shape
    ngrid = N // _BLK

    def body(h_ref, p1_ref, p2_ref, p3_ref, d_ref, w_ref, b_ref,
             raw_ref, sum_ref, sq_ref, acc_s, acc_q):
        i = pl.program_id(0)

        @pl.when(i == 0)
        def _():
            acc_s[...] = jnp.zeros_like(acc_s)
            acc_q[...] = jnp.zeros_like(acc_q)

        d = d_ref[...]
        h1 = d * (p1_ref[0] + p1_ref[1])
        h2 = d * (p2_ref[0] + p2_ref[1])
        h3 = d * (p3_ref[0] + p3_ref[1])
        out = (_dot(h_ref[...], w_ref[0]) + _dot(h1, w_ref[1])
               + _dot(h2, w_ref[2]) + _dot(h3, w_ref[3]) + b_ref[...])
        raw_ref[...] = out
        acc_s[...] += jnp.sum(out, axis=0, keepdims=True)
        acc_q[...] += jnp.sum(out * out, axis=0, keepdims=True)

        @pl.when(i == ngrid - 1)
        def _():
            sum_ref[...] = acc_s[...]
            sq_ref[...] = acc_q[...]

    return pl.pallas_call(
        body,
        grid=(ngrid,),
        in_specs=[
            pl.BlockSpec((_BLK, H), lambda i: (i, 0)),
            pl.BlockSpec((2, _BLK, H), lambda i: (0, i, 0)),
            pl.BlockSpec((2, _BLK, H), lambda i: (0, i, 0)),
            pl.BlockSpec((2, _BLK, H), lambda i: (0, i, 0)),
            pl.BlockSpec((_BLK, H), lambda i: (i, 0)),
            pl.BlockSpec((4, H, H), lambda i: (0, 0, 0)),
            pl.BlockSpec((1, H), lambda i: (0, 0)),
        ],
        out_specs=[
            pl.BlockSpec((_BLK, H), lambda i: (i, 0)),
            pl.BlockSpec((1, H), lambda i: (0, 0)),
            pl.BlockSpec((1, H), lambda i: (0, 0)),
        ],
        out_shape=[
            jax.ShapeDtypeStruct((N, H), _F32),
            jax.ShapeDtypeStruct((1, H), _F32),
            jax.ShapeDtypeStruct((1, H), _F32),
        ],
        scratch_shapes=[
            pltpu.VMEM((1, H), _F32),
            pltpu.VMEM((1, H), _F32),
        ],
    )(h, p1, p2, p3, dinv, Wstack, bias.reshape(1, H))


def _tc_bn(raw, ssum, ssq, gamma, beta, dinv, want_u):
    N, H = raw.shape
    inv_n = 1.0 / N

    def body(r_ref, s_ref, q_ref, g_ref, be_ref, d_ref, *outs):
        m = s_ref[...] * inv_n
        v = q_ref[...] * inv_n - m * m
        hn = (r_ref[...] - m) * lax.rsqrt(v + 1e-5) * g_ref[...] + be_ref[...]
        hn = jnp.maximum(hn, 0.0)
        outs[0][...] = hn
        if want_u:
            outs[1][...] = d_ref[...] * hn

    nout = 2 if want_u else 1
    return pl.pallas_call(
        body,
        grid=(N // _BLK,),
        in_specs=[
            pl.BlockSpec((_BLK, H), lambda i: (i, 0)),
            pl.BlockSpec((1, H), lambda i: (0, 0)),
            pl.BlockSpec((1, H), lambda i: (0, 0)),
            pl.BlockSpec((1, H), lambda i: (0, 0)),
            pl.BlockSpec((1, H), lambda i: (0, 0)),
            pl.BlockSpec((_BLK, H), lambda i: (i, 0)),
        ],
        out_specs=[pl.BlockSpec((_BLK, H), lambda i: (i, 0))] * nout,
        out_shape=[jax.ShapeDtypeStruct((N, H), _F32)] * nout,
    )(raw, ssum, ssq, gamma.reshape(1, H), beta.reshape(1, H), dinv)


def _tc_pool(h, bidx3, G):
    N, H = h.shape
    ngrid = N // _BLK

    def body(h_ref, b_ref, xs_ref, cn_ref, acc_x, acc_c):
        i = pl.program_id(0)

        @pl.when(i == 0)
        def _():
            acc_x[...] = jnp.zeros_like(acc_x)
            acc_c[...] = jnp.zeros_like(acc_c)

        b = b_ref[0, 0, :]
        gids = lax.broadcasted_iota(jnp.int32, (_BLK, G), 1)
        mask = (b[:, None] == gids).astype(_F32)
        cn = lax.dot_general(mask, jnp.ones((_BLK, H), _F32),
                             (((0,), (0,)), ((), ())),
                             preferred_element_type=_F32, precision=_HIGH)
        xs = lax.dot_general(mask, h_ref[...], (((0,), (0,)), ((), ())),
                             preferred_element_type=_F32, precision=_HIGH)
        acc_x[...] += xs
        acc_c[...] += cn

        @pl.when(i == ngrid - 1)
        def _():
            xs_ref[...] = acc_x[...]
            cn_ref[...] = acc_c[...]

    return pl.pallas_call(
        body,
        grid=(ngrid,),
        in_specs=[
            pl.BlockSpec((_BLK, H), lambda i: (i, 0)),
            pl.BlockSpec((1, 1, _BLK), lambda i: (i, 0, 0)),
        ],
        out_specs=[
            pl.BlockSpec((G, H), lambda i: (0, 0)),
            pl.BlockSpec((G, H), lambda i: (0, 0)),
        ],
        out_shape=[
            jax.ShapeDtypeStruct((G, H), _F32),
            jax.ShapeDtypeStruct((G, H), _F32),
        ],
        scratch_shapes=[
            pltpu.VMEM((G, H), _F32),
            pltpu.VMEM((G, H), _F32),
        ],
    )(h, bidx3)


def _tc_head(xsum, counts, latf_pad, Wlat_pad, b_lat,
             W1m, W1s, W1l, b1, W2, b2, W3_pad):
    G, H = xsum.shape
    H2 = W2.shape[1]

    def body(xs_ref, cn_ref, lf_ref, wl_ref, bl_ref, w1m_ref, w1s_ref,
             w1l_ref, b1_ref, w2_ref, b2_ref, w3_ref, o_ref):
        xs = xs_ref[...]
        cm = jnp.maximum(cn_ref[...], 1.0)
        xmean = xs / cm
        lat = jnp.maximum(_dot(lf_ref[...], wl_ref[...]) + bl_ref[...], 0.0)
        z = (_dot(xmean, w1m_ref[...]) + _dot(xs, w1s_ref[...])
             + _dot(lat, w1l_ref[...]) + b1_ref[...])
        z = jnp.maximum(z, 0.0)
        z = jnp.maximum(_dot(z, w2_ref[...]) + b2_ref[...], 0.0)
        o_ref[...] = _dot(z, w3_ref[...])

    return pl.pallas_call(
        body,
        out_shape=jax.ShapeDtypeStruct((G, H), _F32),
    )(xsum, counts, latf_pad, Wlat_pad, b_lat.reshape(1, H),
      W1m, W1s, W1l, b1.reshape(1, H), W2, b2.reshape(1, H2), W3_pad)


def kernel(x, edge_index, batch_idx, lattice_features, W_enc, b_enc, W_lat,
           b_lat, Wc1, bc1, g1, be1, Wc2, bc2, g2, be2, Wc3, bc3, g3, be3,
           W1, b1, W2, b2, W3, b3):
    N, F = x.shape
    H = W_enc.shape[1]
    G = lattice_features.shape[0]
    src = edge_index[0]
    dst = edge_index[1]
    npad = 8 * _NS
    Np = ((N + npad - 1) // npad) * npad
    rpt = Np // _NS

    ones_rows = jnp.ones((128, H), _F32)
    zeros_h = jnp.zeros((rpt, H), _F32)
    bidx3 = batch_idx.reshape(N // _BLK, 1, _BLK)

    degp = _sc_degree(dst, ones_rows, zeros_h)
    h = _tc_encoder(x, W_enc, b_enc)
    dinv, u = _tc_prep(degp, h)

    for Wc, bc, g, be, last in ((Wc1, bc1, g1, be1, False),
                                (Wc2, bc2, g2, be2, False),
                                (Wc3, bc3, g3, be3, True)):
        parts = []
        for k in range(3):
            p = _sc_propagate(u, src, dst, zeros_h)
            parts.append(p)
            if k < 2:
                u = _tc_upass(p, dinv)
        raw, ssum, ssq = _tc_conv(h, parts[0], parts[1], parts[2], dinv, Wc, bc)
        if last:
            (h,) = _tc_bn(raw, ssum, ssq, g, be, dinv, want_u=False)
        else:
            h, u = _tc_bn(raw, ssum, ssq, g, be, dinv, want_u=True)

    xsum, counts = _tc_pool(h, bidx3, G)

    latf_pad = jnp.pad(lattice_features, ((0, 0), (0, 16 - lattice_features.shape[1])))
    Wlat_pad = jnp.pad(W_lat, ((0, 16 - W_lat.shape[0]), (0, 0)))
    W3_pad = jnp.pad(W3, ((0, 0), (0, H - W3.shape[1])))
    W1m, W1s, W1l = W1[:H], W1[H:2 * H], W1[2 * H:]

    head = _tc_head(xsum, counts, latf_pad, Wlat_pad, b_lat,
                    W1m, W1s, W1l, b1, W2, b2, W3_pad)
    return head[:, 0] + b3[0]
